# Initial kernel scaffold; baseline (speedup 1.0000x reference)
import jax
import jax.numpy as jnp
from jax.experimental import pallas as pl


def kernel(x_pooled, batch, num_nodes):
    raise NotImplementedError("write your pallas kernel here")


# speedup vs baseline: 4.1071x; 4.1071x over previous
import jax
import jax.numpy as jnp
from jax import lax
from jax.experimental import pallas as pl
from jax.experimental.pallas import tpu as pltpu
from jax.experimental.pallas import tpu_sc as plsc

_D = 128
_C = 496
_NW = 32


def _unpool_body(n_chunks, last_base, table_hbm, idx_hbm, out_hbm,
                 idx_v, rows_v, sem):
    wid = lax.axis_index("s") * 2 + lax.axis_index("c")

    def body(i, carry):
        c = wid + i * _NW

        @pl.when(c < n_chunks)
        def _():
            base = jnp.minimum(c * _C, last_base)
            pltpu.sync_copy(idx_hbm.at[pl.ds(base, _C)], idx_v)
            pltpu.async_copy(table_hbm.at[idx_v], rows_v, sem).wait()
            pltpu.sync_copy(rows_v, out_hbm.at[pl.ds(base, _C)])

        return carry

    lax.fori_loop(0, (n_chunks + _NW - 1) // _NW, body, 0)


def kernel(x_pooled, batch, num_nodes):
    del num_nodes
    b = batch.shape[0]
    n_chunks = -(-b // _C)
    last_base = b - _C
    idx = batch.astype(jnp.int32)

    import functools
    f = pl.kernel(
        functools.partial(_unpool_body, n_chunks, last_base),
        mesh=plsc.VectorSubcoreMesh(core_axis_name="c", subcore_axis_name="s"),
        out_type=jax.ShapeDtypeStruct((b, _D), jnp.float32),
        scratch_types=[
            pltpu.VMEM((_C,), jnp.int32),
            pltpu.VMEM((_C, _D), jnp.float32),
            pltpu.SemaphoreType.DMA,
        ],
    )
    return f(x_pooled, idx)


# speedup vs baseline: 4.2234x; 1.0283x over previous
import functools

import jax
import jax.numpy as jnp
from jax import lax
from jax.experimental import pallas as pl
from jax.experimental.pallas import tpu as pltpu
from jax.experimental.pallas import tpu_sc as plsc

_D = 128
_C = 496
_NW = 32


def _unpool_body(n_chunks, last_base, max_j, table_hbm, idx_hbm, out_hbm,
                 idx_v, rows0, rows1, gsem, ssem0, ssem1):
    wid = lax.axis_index("s") * 2 + lax.axis_index("c")

    def chunk_id(j):
        return wid + j * _NW

    def base_of(j):
        return jnp.minimum(chunk_id(j) * _C, last_base)

    rows = (rows0, rows1)
    ssem = (ssem0, ssem1)

    for j in range(max_j):
        p = j % 2

        def step(j=j, p=p):
            base = base_of(j)
            pltpu.sync_copy(idx_hbm.at[pl.ds(base, _C)], idx_v)
            if j >= 2:
                pltpu.make_async_copy(
                    rows[p], out_hbm.at[pl.ds(base_of(j - 2), _C)], ssem[p]
                ).wait()
            pltpu.async_copy(table_hbm.at[idx_v], rows[p], gsem).wait()
            pltpu.async_copy(rows[p], out_hbm.at[pl.ds(base, _C)], ssem[p])

        if j == 0:
            step()
        else:
            pl.when(chunk_id(j) < n_chunks)(step)

    for j in range(max(0, max_j - 3), max_j):
        p = j % 2
        issued = chunk_id(j) < n_chunks
        not_waited = chunk_id(j + 2) >= n_chunks if j + 2 < max_j else True

        def drain(j=j, p=p):
            pltpu.make_async_copy(
                rows[p], out_hbm.at[pl.ds(base_of(j), _C)], ssem[p]
            ).wait()

        pl.when(issued & not_waited)(drain)


def kernel(x_pooled, batch, num_nodes):
    del num_nodes
    b = batch.shape[0]
    n_chunks = -(-b // _C)
    last_base = b - _C
    max_j = -(-n_chunks // _NW)
    idx = batch.astype(jnp.int32)

    f = pl.kernel(
        functools.partial(_unpool_body, n_chunks, last_base, max_j),
        mesh=plsc.VectorSubcoreMesh(core_axis_name="c", subcore_axis_name="s"),
        out_type=jax.ShapeDtypeStruct((b, _D), jnp.float32),
        scratch_types=[
            pltpu.VMEM((_C,), jnp.int32),
            pltpu.VMEM((_C, _D), jnp.float32),
            pltpu.VMEM((_C, _D), jnp.float32),
            pltpu.SemaphoreType.DMA,
            pltpu.SemaphoreType.DMA,
            pltpu.SemaphoreType.DMA,
        ],
    )
    return f(x_pooled, idx)


# speedup vs baseline: 4.6146x; 1.0926x over previous
import functools

import jax
import jax.numpy as jnp
from jax import lax
from jax.experimental import pallas as pl
from jax.experimental.pallas import tpu as pltpu
from jax.experimental.pallas import tpu_sc as plsc

_D = 128
_NW = 32
_C = 448
_JPW = 7
_S = _C * _JPW


def _unpool_body(b, table_hbm, idx_hbm, out_hbm,
                 idx_v, rows0, rows1, gsem0, gsem1, ssem0, ssem1):
    wid = lax.axis_index("s") * 2 + lax.axis_index("c")
    base_w = jnp.minimum(wid * _S, b - _S)

    rows = (rows0, rows1)
    gsem = (gsem0, gsem1)
    ssem = (ssem0, ssem1)

    pltpu.sync_copy(idx_hbm.at[pl.ds(base_w, _S)], idx_v)

    def gather(j):
        p = j % 2
        pltpu.async_copy(
            table_hbm.at[idx_v.at[pl.ds(j * _C, _C)]], rows[p], gsem[p])

    def scatter_start(j):
        p = j % 2
        pltpu.async_copy(
            rows[p], out_hbm.at[pl.ds(base_w + j * _C, _C)], ssem[p])

    def scatter_wait(j):
        p = j % 2
        pltpu.make_async_copy(
            rows[p], out_hbm.at[pl.ds(base_w + j * _C, _C)], ssem[p]).wait()

    def gather_wait(j):
        p = j % 2
        pltpu.make_async_copy(
            table_hbm.at[idx_v.at[pl.ds(j * _C, _C)]], rows[p], gsem[p]
        ).wait()

    gather(0)
    for j in range(_JPW):
        if j >= 1:
            scatter_wait(j - 1)
        if j + 1 < _JPW:
            gather(j + 1)
        gather_wait(j)
        scatter_start(j)
    scatter_wait(_JPW - 1)


def kernel(x_pooled, batch, num_nodes):
    del num_nodes
    b = batch.shape[0]
    idx = batch.astype(jnp.int32)

    f = pl.kernel(
        functools.partial(_unpool_body, b),
        mesh=plsc.VectorSubcoreMesh(core_axis_name="c", subcore_axis_name="s"),
        out_type=jax.ShapeDtypeStruct((b, _D), jnp.float32),
        scratch_types=[
            pltpu.VMEM((_S,), jnp.int32),
            pltpu.VMEM((_C, _D), jnp.float32),
            pltpu.VMEM((_C, _D), jnp.float32),
            pltpu.SemaphoreType.DMA,
            pltpu.SemaphoreType.DMA,
            pltpu.SemaphoreType.DMA,
            pltpu.SemaphoreType.DMA,
        ],
    )
    return f(x_pooled, idx)


# speedup vs baseline: 5.5902x; 1.2114x over previous
import functools

import jax
import jax.numpy as jnp
from jax import lax
from jax.experimental import pallas as pl
from jax.experimental.pallas import tpu as pltpu
from jax.experimental.pallas import tpu_sc as plsc

_D = 128
_NW = 32
_NS = 16
_C = 112
_JPW = 28
_S = _C * _JPW
_TSLICE = 632


def _unpool_body(b, v, table_hbm, idx_hbm, out_hbm,
                 table_sp, idx_v, rows0, rows1, gsem0, gsem1, ssem0, ssem1):
    sid = lax.axis_index("s")
    wid = sid * 2 + lax.axis_index("c")
    base_w = jnp.minimum(wid * _S, b - _S)

    rows = (rows0, rows1)
    gsem = (gsem0, gsem1)
    ssem = (ssem0, ssem1)

    base_t = jnp.minimum(sid * _TSLICE, v - _TSLICE)
    pltpu.async_copy(
        table_hbm.at[pl.ds(base_t, _TSLICE)],
        table_sp.at[pl.ds(base_t, _TSLICE)], gsem0)
    pltpu.sync_copy(idx_hbm.at[pl.ds(base_w, _S)], idx_v)
    pltpu.make_async_copy(
        table_hbm.at[pl.ds(base_t, _TSLICE)],
        table_sp.at[pl.ds(base_t, _TSLICE)], gsem0).wait()
    plsc.subcore_barrier()

    def gather(j):
        p = j % 2
        pltpu.async_copy(
            table_sp.at[idx_v.at[pl.ds(j * _C, _C)]], rows[p], gsem[p])

    def gather_wait(j):
        p = j % 2
        pltpu.make_async_copy(
            table_sp.at[idx_v.at[pl.ds(j * _C, _C)]], rows[p], gsem[p]
        ).wait()

    def scatter_start(j):
        p = j % 2
        pltpu.async_copy(
            rows[p], out_hbm.at[pl.ds(base_w + j * _C, _C)], ssem[p])

    def scatter_wait(j):
        p = j % 2
        pltpu.make_async_copy(
            rows[p], out_hbm.at[pl.ds(base_w + j * _C, _C)], ssem[p]).wait()

    gather(0)
    for j in range(_JPW):
        if j >= 1:
            scatter_wait(j - 1)
        if j + 1 < _JPW:
            gather(j + 1)
        gather_wait(j)
        scatter_start(j)
    scatter_wait(_JPW - 1)


def kernel(x_pooled, batch, num_nodes):
    del num_nodes
    b = batch.shape[0]
    v = x_pooled.shape[0]
    idx = batch.astype(jnp.int32)

    f = pl.kernel(
        functools.partial(_unpool_body, b, v),
        mesh=plsc.VectorSubcoreMesh(core_axis_name="c", subcore_axis_name="s"),
        out_type=jax.ShapeDtypeStruct((b, _D), jnp.float32),
        scratch_types=[
            pltpu.VMEM_SHARED((10000, _D), jnp.float32),
            pltpu.VMEM((_S,), jnp.int32),
            pltpu.VMEM((_C, _D), jnp.float32),
            pltpu.VMEM((_C, _D), jnp.float32),
            pltpu.SemaphoreType.DMA,
            pltpu.SemaphoreType.DMA,
            pltpu.SemaphoreType.DMA,
            pltpu.SemaphoreType.DMA,
        ],
    )
    return f(x_pooled, idx)


# speedup vs baseline: 5.6858x; 1.0171x over previous
import functools

import jax
import jax.numpy as jnp
from jax import lax
from jax.experimental import pallas as pl
from jax.experimental.pallas import tpu as pltpu
from jax.experimental.pallas import tpu_sc as plsc

_D = 128
_NW = 32
_NS = 16
_C = 112
_JPW = 28
_S = _C * _JPW
_TSLICE = 632


def _unpool_body(b, v, table_hbm, idx_hbm, out_hbm,
                 table_sp, idx_v, rows0, rows1, gsem0, gsem1, ssem0, ssem1):
    sid = lax.axis_index("s")
    wid = sid * 2 + lax.axis_index("c")
    base_w = jnp.minimum(wid * _S, b - _S)

    rows = (rows0, rows1)
    gsem = (gsem0, gsem1)
    ssem = (ssem0, ssem1)

    base_t = jnp.minimum(sid * _TSLICE, v - _TSLICE)
    pltpu.async_copy(
        table_hbm.at[pl.ds(base_t, _TSLICE)],
        table_sp.at[pl.ds(base_t, _TSLICE)], gsem0)
    pltpu.sync_copy(idx_hbm.at[pl.ds(base_w, _S)], idx_v)
    pltpu.make_async_copy(
        table_hbm.at[pl.ds(base_t, _TSLICE)],
        table_sp.at[pl.ds(base_t, _TSLICE)], gsem0).wait()
    plsc.subcore_barrier()

    def gather(j, p):
        pltpu.async_copy(
            table_sp.at[idx_v.at[pl.ds(j * _C, _C)]], rows[p], gsem[p])

    def gather_wait(j, p):
        pltpu.make_async_copy(
            table_sp.at[idx_v.at[pl.ds(j * _C, _C)]], rows[p], gsem[p]
        ).wait()

    def scatter_start(j, p):
        pltpu.async_copy(
            rows[p], out_hbm.at[pl.ds(base_w + j * _C, _C)], ssem[p])

    def scatter_wait(j, p):
        pltpu.make_async_copy(
            rows[p], out_hbm.at[pl.ds(base_w + j * _C, _C)], ssem[p]).wait()

    gather(0, 0)

    def body(jj, carry):
        for t in range(2):
            j = jj * 2 + t

            @pl.when(j >= 1)
            def _(j=j, t=t):
                scatter_wait(j - 1, 1 - t)

            @pl.when(j + 1 < _JPW)
            def _(j=j, t=t):
                gather(j + 1, 1 - t)

            gather_wait(j, t)
            scatter_start(j, t)
        return carry

    lax.fori_loop(0, _JPW // 2, body, 0)
    scatter_wait(_JPW - 1, (_JPW - 1) % 2)


def kernel(x_pooled, batch, num_nodes):
    del num_nodes
    b = batch.shape[0]
    v = x_pooled.shape[0]
    idx = batch.astype(jnp.int32)

    f = pl.kernel(
        functools.partial(_unpool_body, b, v),
        mesh=plsc.VectorSubcoreMesh(core_axis_name="c", subcore_axis_name="s"),
        out_type=jax.ShapeDtypeStruct((b, _D), jnp.float32),
        scratch_types=[
            pltpu.VMEM_SHARED((10000, _D), jnp.float32),
            pltpu.VMEM((_S,), jnp.int32),
            pltpu.VMEM((_C, _D), jnp.float32),
            pltpu.VMEM((_C, _D), jnp.float32),
            pltpu.SemaphoreType.DMA,
            pltpu.SemaphoreType.DMA,
            pltpu.SemaphoreType.DMA,
            pltpu.SemaphoreType.DMA,
        ],
    )
    return f(x_pooled, idx)


# speedup vs baseline: 5.7960x; 1.0194x over previous
import functools

import jax
import jax.numpy as jnp
from jax import lax
from jax.experimental import pallas as pl
from jax.experimental.pallas import tpu as pltpu
from jax.experimental.pallas import tpu_sc as plsc

_D = 128
_NW = 32
_NS = 16
_C = 112
_JPW = 28
_S = _C * _JPW
_TSLICE = 632
_KH = 4


def _unpool_body(b, v, table_hbm, idx_hbm, out_hbm,
                 table_sp, idx_v, rows0, rows1,
                 gsem0, gsem1, ssem0, ssem1, tsem):
    sid = lax.axis_index("s")
    wid = sid * 2 + lax.axis_index("c")
    base_w = jnp.minimum(wid * _S, b - _S)

    rows = (rows0, rows1)
    gsem = (gsem0, gsem1)
    ssem = (ssem0, ssem1)

    base_t = jnp.minimum(sid * _TSLICE, v - _TSLICE)
    pltpu.async_copy(
        table_hbm.at[pl.ds(base_t, _TSLICE)],
        table_sp.at[pl.ds(base_t, _TSLICE)], tsem)
    pltpu.sync_copy(idx_hbm.at[pl.ds(base_w, _S)], idx_v)

    def gather_h(j, p):
        pltpu.async_copy(
            table_hbm.at[idx_v.at[pl.ds(j * _C, _C)]], rows[p], gsem[p])

    def gather_h_wait(j, p):
        pltpu.make_async_copy(
            table_hbm.at[idx_v.at[pl.ds(j * _C, _C)]], rows[p], gsem[p]
        ).wait()

    def gather(j, p):
        pltpu.async_copy(
            table_sp.at[idx_v.at[pl.ds(j * _C, _C)]], rows[p], gsem[p])

    def gather_wait(j, p):
        pltpu.make_async_copy(
            table_sp.at[idx_v.at[pl.ds(j * _C, _C)]], rows[p], gsem[p]
        ).wait()

    def scatter_start(j, p):
        pltpu.async_copy(
            rows[p], out_hbm.at[pl.ds(base_w + j * _C, _C)], ssem[p])

    def scatter_wait(j, p):
        pltpu.make_async_copy(
            rows[p], out_hbm.at[pl.ds(base_w + j * _C, _C)], ssem[p]).wait()

    gather_h(0, 0)
    for j in range(_KH):
        if j >= 1:
            scatter_wait(j - 1, (j - 1) % 2)
        nx = j + 1
        if nx < _KH:
            gather_h(nx, nx % 2)
        elif nx == _KH:
            pltpu.make_async_copy(
                table_hbm.at[pl.ds(base_t, _TSLICE)],
                table_sp.at[pl.ds(base_t, _TSLICE)], tsem).wait()
            plsc.subcore_barrier()
            gather(nx, nx % 2)
        gather_h_wait(j, j % 2)
        scatter_start(j, j % 2)

    def body(jj, carry):
        for t in range(2):
            j = jj * 2 + t

            scatter_wait(j - 1, 1 - t)

            @pl.when(j + 1 < _JPW)
            def _(j=j, t=t):
                gather(j + 1, 1 - t)

            gather_wait(j, t)
            scatter_start(j, t)
        return carry

    lax.fori_loop(_KH // 2, _JPW // 2, body, 0)
    scatter_wait(_JPW - 1, (_JPW - 1) % 2)


def kernel(x_pooled, batch, num_nodes):
    del num_nodes
    b = batch.shape[0]
    v = x_pooled.shape[0]
    idx = batch.astype(jnp.int32)

    f = pl.kernel(
        functools.partial(_unpool_body, b, v),
        mesh=plsc.VectorSubcoreMesh(core_axis_name="c", subcore_axis_name="s"),
        out_type=jax.ShapeDtypeStruct((b, _D), jnp.float32),
        scratch_types=[
            pltpu.VMEM_SHARED((10000, _D), jnp.float32),
            pltpu.VMEM((_S,), jnp.int32),
            pltpu.VMEM((_C, _D), jnp.float32),
            pltpu.VMEM((_C, _D), jnp.float32),
            pltpu.SemaphoreType.DMA,
            pltpu.SemaphoreType.DMA,
            pltpu.SemaphoreType.DMA,
            pltpu.SemaphoreType.DMA,
            pltpu.SemaphoreType.DMA,
        ],
    )
    return f(x_pooled, idx)
